# baseline (speedup 1.0000x reference)
import functools

import jax
import jax.numpy as jnp
from jax import lax
from jax.experimental import pallas as pl
from jax.experimental.pallas import tpu as pltpu
from jax.experimental.pallas import tpu_sc as plsc


def _gather_kernel(B, D, b_per_w, CH, n_ch, NC):
    mesh = plsc.VectorSubcoreMesh(core_axis_name="c", subcore_axis_name="s")

    @functools.partial(
        pl.kernel,
        mesh=mesh,
        compiler_params=pltpu.CompilerParams(use_tc_tiling_on_sc=False),
        out_type=jax.ShapeDtypeStruct((B, D), jnp.float32),
        scratch_types=[
            pltpu.VMEM((CH,), jnp.int32),
            pltpu.VMEM((CH, D), jnp.float32),
            pltpu.SemaphoreType.DMA,
        ],
    )
    def k(idx_hbm, table_hbm, out_hbm, idx_v, rows_v, sem):
        wid = lax.axis_index("s") * NC + lax.axis_index("c")
        w_base = wid * b_per_w

        def body(i, carry):
            base = w_base + i * CH
            pltpu.sync_copy(idx_hbm.at[pl.ds(base, CH)], idx_v)
            pltpu.async_copy(table_hbm.at[idx_v], rows_v, sem).wait()
            pltpu.sync_copy(rows_v, out_hbm.at[pl.ds(base, CH)])
            return carry

        lax.fori_loop(0, n_ch, body, 0)

    return k


def kernel(x, table):
    B = x.shape[0] * x.shape[1]
    D = table.shape[1]
    NW = 32
    NC = 2
    b_per_w = B // NW
    CH = 128
    n_ch = b_per_w // CH
    xf = x.reshape(B).astype(jnp.int32)
    out = _gather_kernel(B, D, b_per_w, CH, n_ch, NC)(xf, table)
    return out.reshape(x.shape[0], x.shape[1], D)


# speedup vs baseline: 1.1947x; 1.1947x over previous
import functools

import jax
import jax.numpy as jnp
from jax import lax
from jax.experimental import pallas as pl
from jax.experimental.pallas import tpu as pltpu
from jax.experimental.pallas import tpu_sc as plsc

_NW = 32
_NC = 2
_CH = 128
_NGC = 4


def _gather_kernel(n_chunks, D, ch_per_w, n_groups):
    mesh = plsc.VectorSubcoreMesh(core_axis_name="c", subcore_axis_name="s")

    @functools.partial(
        pl.kernel,
        mesh=mesh,
        compiler_params=pltpu.CompilerParams(use_tc_tiling_on_sc=False),
        out_type=jax.ShapeDtypeStruct((n_chunks, _CH, D), jnp.float32),
        scratch_types=[
            pltpu.VMEM((2, _NGC, _CH), jnp.int32),
            pltpu.VMEM((2, _NGC, _CH, D), jnp.float32),
            pltpu.SemaphoreType.DMA,
            pltpu.SemaphoreType.DMA,
            pltpu.SemaphoreType.DMA,
            pltpu.SemaphoreType.DMA,
        ],
    )
    def k(idx_hbm, table_hbm, out_hbm, idx_v, rows_v, sg0, sg1, sw0, sw1):
        wid = lax.axis_index("s") * _NC + lax.axis_index("c")
        w_cb = wid * ch_per_w
        sem_g = (sg0, sg1)
        sem_w = (sw0, sw1)

        def run_group(g, b, other):
            @pl.when(g >= 2)
            def _():
                pltpu.make_async_copy(
                    rows_v.at[b],
                    out_hbm.at[pl.ds(w_cb + (g - 2) * _NGC, _NGC)],
                    sem_w[b],
                ).wait()

            descs = [
                pltpu.async_copy(
                    table_hbm.at[idx_v.at[b, j]], rows_v.at[b, j], sem_g[b]
                )
                for j in range(_NGC)
            ]

            @pl.when(g + 1 < n_groups)
            def _():
                pltpu.sync_copy(
                    idx_hbm.at[pl.ds(w_cb + (g + 1) * _NGC, _NGC)],
                    idx_v.at[other],
                )

            for d in descs:
                d.wait()
            pltpu.async_copy(
                rows_v.at[b],
                out_hbm.at[pl.ds(w_cb + g * _NGC, _NGC)],
                sem_w[b],
            )

        pltpu.sync_copy(idx_hbm.at[pl.ds(w_cb, _NGC)], idx_v.at[0])

        @pl.loop(0, n_groups, step=2)
        def _(gbase):
            run_group(gbase, 0, 1)
            run_group(gbase + 1, 1, 0)

        for g, b in ((n_groups - 2, 0), (n_groups - 1, 1)):
            pltpu.make_async_copy(
                rows_v.at[b],
                out_hbm.at[pl.ds(w_cb + g * _NGC, _NGC)],
                sem_w[b],
            ).wait()

    return k


def kernel(x, table):
    B = x.shape[0] * x.shape[1]
    D = table.shape[1]
    n_chunks = B // _CH
    ch_per_w = n_chunks // _NW
    n_groups = ch_per_w // _NGC
    xf = x.reshape(n_chunks, _CH).astype(jnp.int32)
    out = _gather_kernel(n_chunks, D, ch_per_w, n_groups)(xf, table)
    return out.reshape(x.shape[0], x.shape[1], D)
